# baseline (speedup 1.0000x reference)
import functools

import jax
import jax.numpy as jnp
from jax import lax
from jax.experimental import pallas as pl
from jax.experimental.pallas import tpu as pltpu
from jax.experimental.pallas import tpu_sc as plsc

N_NODES = 10000
N_EDGES = 320000
D_IN = 128
D_HID = 128
D_OUT = 10
N_GRAPHS = 64

NC = 2
NS = 16
CHUNK = 128
NW = NC * NS
E_PAD = 327680
T = E_PAD // (NW * CHUNK)
SG = 16
NSG = T // SG
ROW_BLK = 1000
N_BLKS = N_NODES // ROW_BLK
ROW_BLK = 1000
N_BLKS = N_NODES // ROW_BLK

_sc_mesh = plsc.VectorSubcoreMesh(
    core_axis_name="c", subcore_axis_name="s", num_cores=NC, num_subcores=NS)


@functools.partial(
    pl.kernel,
    out_type=jax.ShapeDtypeStruct((NC, N_NODES), jnp.float32),
    mesh=_sc_mesh,
    scratch_types=[
        pltpu.VMEM_SHARED((N_NODES,), jnp.float32),
        pltpu.VMEM((SG, CHUNK), jnp.int32),
        pltpu.VMEM((SG, CHUNK), jnp.float32),
        pltpu.SemaphoreType.DMA,
        pltpu.SemaphoreType.DMA,
    ],
)
def _sc_deg(col3_hbm, ew3_hbm, zeros_hbm, degp_hbm,
            deg_sh, col_v, ew_v, sem_i, sem_s):
    c = lax.axis_index("c")
    s = lax.axis_index("s")
    w = c * NS + s

    @pl.when(s == 0)
    def _():
        pltpu.sync_copy(zeros_hbm, deg_sh)

    plsc.subcore_barrier()

    def wait_scat(_, x):
        pltpu.make_async_copy(
            ew_v.at[0], deg_sh.at[col_v.at[0]], sem_s).wait()
        return x

    def body(k, _):
        m = k % SG

        @pl.when(m == 0)
        def _():
            @pl.when(k > 0)
            def _():
                lax.fori_loop(0, SG, wait_scat, 0)
            g = k // SG
            pltpu.async_copy(
                col3_hbm.at[w, pl.ds(g * SG, SG), :], col_v, sem_i).wait()
            pltpu.async_copy(
                ew3_hbm.at[w, pl.ds(g * SG, SG), :], ew_v, sem_i).wait()

        pltpu.async_copy(
            ew_v.at[m], deg_sh.at[col_v.at[m]], sem_s, add=True)
        return 0

    lax.fori_loop(0, T, body, 0)
    lax.fori_loop(0, SG, wait_scat, 0)
    plsc.subcore_barrier()

    @pl.when(s == 0)
    def _():
        pltpu.sync_copy(deg_sh, degp_hbm.at[c])


@functools.partial(
    pl.kernel,
    out_type=jax.ShapeDtypeStruct((NC, N_NODES, D_HID), jnp.float32),
    mesh=_sc_mesh,
    scratch_types=[
        pltpu.VMEM_SHARED((N_NODES, D_HID), jnp.float32),
        pltpu.VMEM((SG, CHUNK), jnp.int32),
        pltpu.VMEM((SG, CHUNK), jnp.int32),
        pltpu.VMEM((SG, CHUNK), jnp.float32),
        pltpu.VMEM((2, CHUNK, D_HID), jnp.float32),
        pltpu.SemaphoreType.DMA,
        pltpu.SemaphoreType.DMA((2,)),
        pltpu.SemaphoreType.DMA((2,)),
    ],
)
def _sc_agg(y_hbm, row3_hbm, col3_hbm, ew3_hbm, zeros_hbm, aggp_hbm,
            accum_sh, row_v, col_v, ew_v, rows_v, sem_i, sem_g, sem_s):
    c = lax.axis_index("c")
    s = lax.axis_index("s")
    w = c * NS + s

    @pl.when(jnp.logical_and(s == 0, c == 0))
    def _():
        pltpu.sync_copy(y_hbm, accum_sh)

    @pl.when(jnp.logical_and(s == 0, c == 1))
    def _():
        pltpu.sync_copy(zeros_hbm, accum_sh)

    plsc.subcore_barrier()

    def wait_scat(b):
        pltpu.make_async_copy(
            rows_v.at[b], accum_sh.at[col_v.at[0]], sem_s.at[b]).wait()

    def body(k, _):
        @pl.when(k >= 1)
        def _():
            b1 = (k - 1) % 2
            m1 = (k - 1) % SG
            pltpu.make_async_copy(
                y_hbm.at[row_v.at[0]], rows_v.at[b1], sem_g.at[b1]).wait()

            def scale(g2, _):
                wvec = ew_v[m1, pl.ds(g2 * 16, 16)]
                for i in range(16):
                    wi = wvec[i]
                    r = g2 * 16 + i
                    for j in range(D_HID // 16):
                        sl = pl.ds(j * 16, 16)
                        rows_v[b1, r, sl] = rows_v[b1, r, sl] * wi
                return 0

            lax.fori_loop(0, CHUNK // 16, scale, 0)
            pltpu.async_copy(
                rows_v.at[b1], accum_sh.at[col_v.at[m1]], sem_s.at[b1],
                add=True)

        @pl.when(k < T)
        def _():
            m = k % SG
            b = k % 2

            @pl.when(m == 0)
            def _():
                @pl.when(k >= 1)
                def _():
                    wait_scat((k - 1) % 2)

                @pl.when(k >= 2)
                def _():
                    wait_scat(k % 2)

                g = k // SG
                pltpu.async_copy(
                    row3_hbm.at[w, pl.ds(g * SG, SG), :], row_v, sem_i).wait()
                pltpu.async_copy(
                    col3_hbm.at[w, pl.ds(g * SG, SG), :], col_v, sem_i).wait()
                pltpu.async_copy(
                    ew3_hbm.at[w, pl.ds(g * SG, SG), :], ew_v, sem_i).wait()

            @pl.when(jnp.logical_and(k >= 2, m >= 2))
            def _():
                wait_scat(b)

            pltpu.async_copy(
                y_hbm.at[row_v.at[m]], rows_v.at[b], sem_g.at[b])

        return 0

    lax.fori_loop(0, T + 1, body, 0)
    wait_scat(0)
    wait_scat(1)
    plsc.subcore_barrier()

    @pl.when(s == 0)
    def _():
        pltpu.sync_copy(accum_sh, aggp_hbm.at[c])


def _dinv_of(degp_ref):
    deg = degp_ref[0, 0, :] + degp_ref[0, 1, :] + 1.0
    return lax.rsqrt(deg)


def _tc_y1_body(x_ref, w_ref, degp_ref, y_ref):
    dinv = _dinv_of(degp_ref)
    y_ref[...] = jnp.dot(x_ref[...], w_ref[...],
                         preferred_element_type=jnp.float32) * dinv[:, None]


def _tc_y1(x, W1, degp):
    return pl.pallas_call(
        _tc_y1_body,
        grid=(N_BLKS,),
        in_specs=[
            pl.BlockSpec((ROW_BLK, D_IN), lambda i: (i, 0)),
            pl.BlockSpec((D_IN, D_HID), lambda i: (0, 0)),
            pl.BlockSpec((1, NC, ROW_BLK), lambda i: (i, 0, 0)),
        ],
        out_specs=pl.BlockSpec((ROW_BLK, D_HID), lambda i: (i, 0)),
        out_shape=jax.ShapeDtypeStruct((N_NODES, D_HID), jnp.float32),
    )(x, W1, degp)


def _tc_y2_body(aggp_ref, degp_ref, b_ref, w_ref, y_ref):
    dinv = _dinv_of(degp_ref)
    agg = aggp_ref[0, :, :] + aggp_ref[1, :, :]
    h = jnp.maximum(agg * dinv[:, None] + b_ref[...], 0.0)
    y_ref[...] = jnp.dot(h, w_ref[...],
                         preferred_element_type=jnp.float32) * dinv[:, None]


def _tc_y2(aggp, degp, b1, W2):
    return pl.pallas_call(
        _tc_y2_body,
        grid=(N_BLKS,),
        in_specs=[
            pl.BlockSpec((NC, ROW_BLK, D_HID), lambda i: (0, i, 0)),
            pl.BlockSpec((1, NC, ROW_BLK), lambda i: (i, 0, 0)),
            pl.BlockSpec((1, D_HID), lambda i: (0, 0)),
            pl.BlockSpec((D_HID, D_HID), lambda i: (0, 0)),
        ],
        out_specs=pl.BlockSpec((ROW_BLK, D_HID), lambda i: (i, 0)),
        out_shape=jax.ShapeDtypeStruct((N_NODES, D_HID), jnp.float32),
    )(aggp, degp, b1, W2)


def _tc_final_body(aggp_ref, degp_ref, b_ref, batch_ref, wl_ref, bl_ref,
                   out_ref, psum, pcnt):
    i = pl.program_id(0)

    @pl.when(i == 0)
    def _():
        psum[...] = jnp.zeros_like(psum)
        pcnt[...] = jnp.zeros_like(pcnt)

    dinv = _dinv_of(degp_ref)
    agg = aggp_ref[0, :, :] + aggp_ref[1, :, :]
    h = jnp.maximum(agg * dinv[:, None] + b_ref[...], 0.0)
    seg = batch_ref[0, :, :]
    gids = lax.broadcasted_iota(jnp.int32, (N_GRAPHS, ROW_BLK), 0)
    onehot = jnp.where(gids == seg, 1.0, 0.0)
    psum[...] += jnp.dot(onehot, h, preferred_element_type=jnp.float32)
    pcnt[...] += jnp.sum(onehot, axis=1, keepdims=True)

    @pl.when(i == N_BLKS - 1)
    def _():
        pooled = psum[...] / jnp.maximum(pcnt[...], 1.0)
        out_ref[...] = jnp.dot(pooled, wl_ref[...],
                               preferred_element_type=jnp.float32) + bl_ref[...]


def _tc_final(aggp, degp, b2, batch3, Wlin, blin):
    return pl.pallas_call(
        _tc_final_body,
        grid=(N_BLKS,),
        in_specs=[
            pl.BlockSpec((NC, ROW_BLK, D_HID), lambda i: (0, i, 0)),
            pl.BlockSpec((1, NC, ROW_BLK), lambda i: (i, 0, 0)),
            pl.BlockSpec((1, D_HID), lambda i: (0, 0)),
            pl.BlockSpec((1, 1, ROW_BLK), lambda i: (i, 0, 0)),
            pl.BlockSpec((D_HID, D_OUT), lambda i: (0, 0)),
            pl.BlockSpec((1, D_OUT), lambda i: (0, 0)),
        ],
        out_specs=pl.BlockSpec((N_GRAPHS, D_OUT), lambda i: (0, 0)),
        out_shape=jax.ShapeDtypeStruct((N_GRAPHS, D_OUT), jnp.float32),
        scratch_shapes=[
            pltpu.VMEM((N_GRAPHS, D_HID), jnp.float32),
            pltpu.VMEM((N_GRAPHS, 1), jnp.float32),
        ],
    )(aggp, degp, b2, batch3, Wlin, blin)


@jax.jit
def kernel(x, edge_index, edge_attr, batch, W1, b1, W2, b2, Wlin, blin):
    row = edge_index[0].astype(jnp.int32)
    col = edge_index[1].astype(jnp.int32)
    ew = edge_attr.astype(jnp.float32)
    batch3 = batch.astype(jnp.int32).reshape(N_BLKS, 1, ROW_BLK)
    zeros1 = jnp.zeros((N_NODES,), jnp.float32)
    zeros2 = jnp.zeros((N_NODES, D_HID), jnp.float32)

    pad = E_PAD - N_EDGES
    padidx = jnp.arange(pad, dtype=jnp.int32) % N_NODES
    row3 = jnp.concatenate([row, padidx]).reshape(NW, T, CHUNK)
    col3 = jnp.concatenate([col, padidx]).reshape(NW, T, CHUNK)
    ew3 = jnp.concatenate(
        [ew, jnp.zeros((pad,), jnp.float32)]).reshape(NW, T, CHUNK)

    degp = _sc_deg(col3, ew3, zeros1)
    degp3 = degp.reshape(NC, N_BLKS, ROW_BLK).transpose(1, 0, 2)
    y1 = _tc_y1(x, W1, degp3)
    aggp1 = _sc_agg(y1, row3, col3, ew3, zeros2)
    y2 = _tc_y2(aggp1, degp3, b1.reshape(1, D_HID), W2)
    aggp2 = _sc_agg(y2, row3, col3, ew3, zeros2)
    return _tc_final(aggp2, degp3, b2.reshape(1, D_HID), batch3,
                     Wlin, blin.reshape(1, D_OUT))


# speedup vs baseline: 1.2155x; 1.2155x over previous
import functools

import jax
import jax.numpy as jnp
from jax import lax
from jax.experimental import pallas as pl
from jax.experimental.pallas import tpu as pltpu
from jax.experimental.pallas import tpu_sc as plsc

N_NODES = 10000
N_EDGES = 320000
D_IN = 128
D_HID = 128
D_OUT = 10
N_GRAPHS = 64

NC = 2
NS = 16
CHUNK = 64
NW = NC * NS
T = 160
E_PAD = NW * T * CHUNK
SG = 16
NBUF = 4
ROW_BLK = 1000
N_BLKS = N_NODES // ROW_BLK
ROW_BLK = 1000
N_BLKS = N_NODES // ROW_BLK

_sc_mesh = plsc.VectorSubcoreMesh(
    core_axis_name="c", subcore_axis_name="s", num_cores=NC, num_subcores=NS)


@functools.partial(
    pl.kernel,
    out_type=jax.ShapeDtypeStruct((NC, N_NODES), jnp.float32),
    mesh=_sc_mesh,
    scratch_types=[
        pltpu.VMEM_SHARED((N_NODES,), jnp.float32),
        pltpu.VMEM((SG, CHUNK), jnp.int32),
        pltpu.VMEM((SG, CHUNK), jnp.float32),
        pltpu.SemaphoreType.DMA,
        pltpu.SemaphoreType.DMA,
    ],
)
def _sc_deg(col3_hbm, ew3_hbm, zeros_hbm, degp_hbm,
            deg_sh, col_v, ew_v, sem_i, sem_s):
    c = lax.axis_index("c")
    s = lax.axis_index("s")
    w = c * NS + s

    @pl.when(s == 0)
    def _():
        pltpu.sync_copy(zeros_hbm, deg_sh)

    plsc.subcore_barrier()

    def wait_scat(_, x):
        pltpu.make_async_copy(
            ew_v.at[0], deg_sh.at[col_v.at[0]], sem_s).wait()
        return x

    def body(k, _):
        m = k % SG

        @pl.when(m == 0)
        def _():
            @pl.when(k > 0)
            def _():
                lax.fori_loop(0, SG, wait_scat, 0)
            g = k // SG
            pltpu.async_copy(
                col3_hbm.at[w, pl.ds(g * SG, SG), :], col_v, sem_i).wait()
            pltpu.async_copy(
                ew3_hbm.at[w, pl.ds(g * SG, SG), :], ew_v, sem_i).wait()

        pltpu.async_copy(
            ew_v.at[m], deg_sh.at[col_v.at[m]], sem_s, add=True)
        return 0

    lax.fori_loop(0, T, body, 0)
    lax.fori_loop(0, SG, wait_scat, 0)
    plsc.subcore_barrier()

    @pl.when(s == 0)
    def _():
        pltpu.sync_copy(deg_sh, degp_hbm.at[c])


@functools.partial(
    pl.kernel,
    out_type=jax.ShapeDtypeStruct((NC, N_NODES, D_HID), jnp.float32),
    mesh=_sc_mesh,
    scratch_types=[
        pltpu.VMEM_SHARED((N_NODES, D_HID), jnp.float32),
        pltpu.VMEM((SG, CHUNK), jnp.int32),
        pltpu.VMEM((SG, CHUNK), jnp.int32),
        pltpu.VMEM((SG, CHUNK), jnp.float32),
        pltpu.VMEM((NBUF, CHUNK, D_HID), jnp.float32),
        pltpu.SemaphoreType.DMA,
        pltpu.SemaphoreType.DMA((NBUF,)),
        pltpu.SemaphoreType.DMA((NBUF,)),
    ],
)
def _sc_agg(y_hbm, row3_hbm, col3_hbm, ew3_hbm, zeros_hbm, aggp_hbm,
            accum_sh, row_v, col_v, ew_v, rows_v, sem_i, sem_g, sem_s):
    c = lax.axis_index("c")
    s = lax.axis_index("s")
    w = c * NS + s

    @pl.when(jnp.logical_and(s == 0, c == 0))
    def _():
        pltpu.sync_copy(y_hbm, accum_sh)

    @pl.when(jnp.logical_and(s == 0, c == 1))
    def _():
        pltpu.sync_copy(zeros_hbm, accum_sh)

    plsc.subcore_barrier()

    def wait_scat(b):
        pltpu.make_async_copy(
            rows_v.at[b], accum_sh.at[col_v.at[0]], sem_s.at[b]).wait()

    def process(km1):
        b1 = km1 % NBUF
        m1 = km1 % SG
        pltpu.make_async_copy(
            y_hbm.at[row_v.at[0]], rows_v.at[b1], sem_g.at[b1]).wait()

        def scale(g2, _):
            wvec = ew_v[m1, pl.ds(g2 * 16, 16)]
            for i in range(16):
                wi = wvec[i]
                r = g2 * 16 + i
                for j in range(D_HID // 16):
                    sl = pl.ds(j * 16, 16)
                    rows_v[b1, r, sl] = rows_v[b1, r, sl] * wi
            return 0

        lax.fori_loop(0, CHUNK // 16, scale, 0)
        pltpu.async_copy(
            rows_v.at[b1], accum_sh.at[col_v.at[m1]], sem_s.at[b1],
            add=True)

    def body(k, _):
        m = k % SG
        b = k % NBUF

        @pl.when(m != 0)
        def _():
            @pl.when(jnp.logical_and(k >= NBUF, m >= NBUF))
            def _():
                wait_scat(b)

            pltpu.async_copy(
                y_hbm.at[row_v.at[m]], rows_v.at[b], sem_g.at[b])
            process(k - 1)

        @pl.when(m == 0)
        def _():
            @pl.when(k >= 1)
            def _():
                process(k - 1)

            @pl.when(k < T)
            def _():
                for d in range(1, NBUF + 1):
                    @pl.when(k >= d)
                    def _(d=d):
                        wait_scat((k - d) % NBUF)

                g = k // SG
                pltpu.async_copy(
                    row3_hbm.at[w, pl.ds(g * SG, SG), :], row_v, sem_i).wait()
                pltpu.async_copy(
                    col3_hbm.at[w, pl.ds(g * SG, SG), :], col_v, sem_i).wait()
                pltpu.async_copy(
                    ew3_hbm.at[w, pl.ds(g * SG, SG), :], ew_v, sem_i).wait()
                pltpu.async_copy(
                    y_hbm.at[row_v.at[0]], rows_v.at[b], sem_g.at[b])

        return 0

    lax.fori_loop(0, T + 1, body, 0)
    for b in range(NBUF):
        wait_scat(b)
    plsc.subcore_barrier()

    @pl.when(s == 0)
    def _():
        pltpu.sync_copy(accum_sh, aggp_hbm.at[c])


def _dinv_of(degp_ref):
    deg = degp_ref[0, 0, :] + degp_ref[0, 1, :] + 1.0
    return lax.rsqrt(deg)


def _tc_y1_body(x_ref, w_ref, degp_ref, y_ref):
    dinv = _dinv_of(degp_ref)
    y_ref[...] = jnp.dot(x_ref[...], w_ref[...],
                         preferred_element_type=jnp.float32) * dinv[:, None]


def _tc_y1(x, W1, degp):
    return pl.pallas_call(
        _tc_y1_body,
        grid=(N_BLKS,),
        in_specs=[
            pl.BlockSpec((ROW_BLK, D_IN), lambda i: (i, 0)),
            pl.BlockSpec((D_IN, D_HID), lambda i: (0, 0)),
            pl.BlockSpec((1, NC, ROW_BLK), lambda i: (i, 0, 0)),
        ],
        out_specs=pl.BlockSpec((ROW_BLK, D_HID), lambda i: (i, 0)),
        out_shape=jax.ShapeDtypeStruct((N_NODES, D_HID), jnp.float32),
    )(x, W1, degp)


def _tc_y2_body(aggp_ref, degp_ref, b_ref, w_ref, y_ref):
    dinv = _dinv_of(degp_ref)
    agg = aggp_ref[0, :, :] + aggp_ref[1, :, :]
    h = jnp.maximum(agg * dinv[:, None] + b_ref[...], 0.0)
    y_ref[...] = jnp.dot(h, w_ref[...],
                         preferred_element_type=jnp.float32) * dinv[:, None]


def _tc_y2(aggp, degp, b1, W2):
    return pl.pallas_call(
        _tc_y2_body,
        grid=(N_BLKS,),
        in_specs=[
            pl.BlockSpec((NC, ROW_BLK, D_HID), lambda i: (0, i, 0)),
            pl.BlockSpec((1, NC, ROW_BLK), lambda i: (i, 0, 0)),
            pl.BlockSpec((1, D_HID), lambda i: (0, 0)),
            pl.BlockSpec((D_HID, D_HID), lambda i: (0, 0)),
        ],
        out_specs=pl.BlockSpec((ROW_BLK, D_HID), lambda i: (i, 0)),
        out_shape=jax.ShapeDtypeStruct((N_NODES, D_HID), jnp.float32),
    )(aggp, degp, b1, W2)


def _tc_final_body(aggp_ref, degp_ref, b_ref, batch_ref, wl_ref, bl_ref,
                   out_ref, psum, pcnt):
    i = pl.program_id(0)

    @pl.when(i == 0)
    def _():
        psum[...] = jnp.zeros_like(psum)
        pcnt[...] = jnp.zeros_like(pcnt)

    dinv = _dinv_of(degp_ref)
    agg = aggp_ref[0, :, :] + aggp_ref[1, :, :]
    h = jnp.maximum(agg * dinv[:, None] + b_ref[...], 0.0)
    seg = batch_ref[0, :, :]
    gids = lax.broadcasted_iota(jnp.int32, (N_GRAPHS, ROW_BLK), 0)
    onehot = jnp.where(gids == seg, 1.0, 0.0)
    psum[...] += jnp.dot(onehot, h, preferred_element_type=jnp.float32)
    pcnt[...] += jnp.sum(onehot, axis=1, keepdims=True)

    @pl.when(i == N_BLKS - 1)
    def _():
        pooled = psum[...] / jnp.maximum(pcnt[...], 1.0)
        out_ref[...] = jnp.dot(pooled, wl_ref[...],
                               preferred_element_type=jnp.float32) + bl_ref[...]


def _tc_final(aggp, degp, b2, batch3, Wlin, blin):
    return pl.pallas_call(
        _tc_final_body,
        grid=(N_BLKS,),
        in_specs=[
            pl.BlockSpec((NC, ROW_BLK, D_HID), lambda i: (0, i, 0)),
            pl.BlockSpec((1, NC, ROW_BLK), lambda i: (i, 0, 0)),
            pl.BlockSpec((1, D_HID), lambda i: (0, 0)),
            pl.BlockSpec((1, 1, ROW_BLK), lambda i: (i, 0, 0)),
            pl.BlockSpec((D_HID, D_OUT), lambda i: (0, 0)),
            pl.BlockSpec((1, D_OUT), lambda i: (0, 0)),
        ],
        out_specs=pl.BlockSpec((N_GRAPHS, D_OUT), lambda i: (0, 0)),
        out_shape=jax.ShapeDtypeStruct((N_GRAPHS, D_OUT), jnp.float32),
        scratch_shapes=[
            pltpu.VMEM((N_GRAPHS, D_HID), jnp.float32),
            pltpu.VMEM((N_GRAPHS, 1), jnp.float32),
        ],
    )(aggp, degp, b2, batch3, Wlin, blin)


@jax.jit
def kernel(x, edge_index, edge_attr, batch, W1, b1, W2, b2, Wlin, blin):
    row = edge_index[0].astype(jnp.int32)
    col = edge_index[1].astype(jnp.int32)
    ew = edge_attr.astype(jnp.float32)
    batch3 = batch.astype(jnp.int32).reshape(N_BLKS, 1, ROW_BLK)
    zeros1 = jnp.zeros((N_NODES,), jnp.float32)
    zeros2 = jnp.zeros((N_NODES, D_HID), jnp.float32)

    pad = E_PAD - N_EDGES
    padidx = jnp.arange(pad, dtype=jnp.int32) % N_NODES
    row3 = jnp.concatenate([row, padidx]).reshape(NW, T, CHUNK)
    col3 = jnp.concatenate([col, padidx]).reshape(NW, T, CHUNK)
    ew3 = jnp.concatenate(
        [ew, jnp.zeros((pad,), jnp.float32)]).reshape(NW, T, CHUNK)

    degp = _sc_deg(col3, ew3, zeros1)
    degp3 = degp.reshape(NC, N_BLKS, ROW_BLK).transpose(1, 0, 2)
    y1 = _tc_y1(x, W1, degp3)
    aggp1 = _sc_agg(y1, row3, col3, ew3, zeros2)
    y2 = _tc_y2(aggp1, degp3, b1.reshape(1, D_HID), W2)
    aggp2 = _sc_agg(y2, row3, col3, ew3, zeros2)
    return _tc_final(aggp2, degp3, b2.reshape(1, D_HID), batch3,
                     Wlin, blin.reshape(1, D_OUT))


# speedup vs baseline: 2.3687x; 1.9487x over previous
import functools

import jax
import jax.numpy as jnp
from jax import lax
from jax.experimental import pallas as pl
from jax.experimental.pallas import tpu as pltpu
from jax.experimental.pallas import tpu_sc as plsc

N_NODES = 10000
N_EDGES = 320000
D_IN = 128
D_HID = 128
D_OUT = 10
N_GRAPHS = 64

NC = 2
NS = 16
CHUNK = 32
NW = NC * NS
T = 320
E_PAD = NW * T * CHUNK
G = 8
NG = T // G
ROW_BLK = 1000
N_BLKS = N_NODES // ROW_BLK
ROW_BLK = 1000
N_BLKS = N_NODES // ROW_BLK

_sc_mesh = plsc.VectorSubcoreMesh(
    core_axis_name="c", subcore_axis_name="s", num_cores=NC, num_subcores=NS)


@functools.partial(
    pl.kernel,
    out_type=jax.ShapeDtypeStruct((NC, N_NODES), jnp.float32),
    mesh=_sc_mesh,
    scratch_types=[
        pltpu.VMEM_SHARED((N_NODES,), jnp.float32),
        pltpu.VMEM((G, CHUNK), jnp.int32),
        pltpu.VMEM((G, CHUNK), jnp.float32),
        pltpu.SemaphoreType.DMA,
        pltpu.SemaphoreType.DMA((G,)),
    ],
)
def _sc_deg(col3_hbm, ew3_hbm, zeros_hbm, degp_hbm,
            deg_sh, col_v, ew_v, sem_i, sem_s):
    c = lax.axis_index("c")
    s = lax.axis_index("s")
    w = c * NS + s

    @pl.when(s == 0)
    def _():
        pltpu.sync_copy(zeros_hbm, deg_sh)

    plsc.subcore_barrier()

    def drain(_t, x):
        for b in range(G):
            pltpu.make_async_copy(
                ew_v.at[b], deg_sh.at[col_v.at[b]], sem_s.at[b]).wait()
        return x

    def body(t, _):
        @pl.when(t > 0)
        def _():
            drain(t, 0)
        di = pltpu.async_copy(
            col3_hbm.at[w, pl.ds(t * G, G), :], col_v, sem_i)
        dw = pltpu.async_copy(
            ew3_hbm.at[w, pl.ds(t * G, G), :], ew_v, sem_i)
        di.wait()
        dw.wait()
        for b in range(G):
            pltpu.async_copy(
                ew_v.at[b], deg_sh.at[col_v.at[b]], sem_s.at[b], add=True)
        return 0

    lax.fori_loop(0, NG, body, 0)
    drain(0, 0)
    plsc.subcore_barrier()

    @pl.when(s == 0)
    def _():
        pltpu.sync_copy(deg_sh, degp_hbm.at[c])


@functools.partial(
    pl.kernel,
    out_type=jax.ShapeDtypeStruct((NC, N_NODES, D_HID), jnp.float32),
    mesh=_sc_mesh,
    scratch_types=[
        pltpu.VMEM_SHARED((N_NODES, D_HID), jnp.float32),
        pltpu.VMEM((G, CHUNK), jnp.int32),
        pltpu.VMEM((G, CHUNK), jnp.int32),
        pltpu.VMEM((G, CHUNK), jnp.float32),
        pltpu.VMEM((G, CHUNK, D_HID), jnp.float32),
        pltpu.SemaphoreType.DMA,
        pltpu.SemaphoreType.DMA((G,)),
        pltpu.SemaphoreType.DMA((G,)),
    ],
)
def _sc_agg(y_hbm, row3_hbm, col3_hbm, ew3_hbm, zeros_hbm, aggp_hbm,
            accum_sh, row_v, col_v, ew_v, rows_v, sem_i, sem_g, sem_s):
    c = lax.axis_index("c")
    s = lax.axis_index("s")
    w = c * NS + s

    @pl.when(jnp.logical_and(s == 0, c == 0))
    def _():
        pltpu.sync_copy(y_hbm, accum_sh)

    @pl.when(jnp.logical_and(s == 0, c == 1))
    def _():
        pltpu.sync_copy(zeros_hbm, accum_sh)

    plsc.subcore_barrier()

    def drain(_t, x):
        for b in range(G):
            pltpu.make_async_copy(
                rows_v.at[b], accum_sh.at[col_v.at[b]], sem_s.at[b]).wait()
        return x

    def body(t, _):
        @pl.when(t > 0)
        def _():
            drain(t, 0)
        di = pltpu.async_copy(
            row3_hbm.at[w, pl.ds(t * G, G), :], row_v, sem_i)
        dc = pltpu.async_copy(
            col3_hbm.at[w, pl.ds(t * G, G), :], col_v, sem_i)
        dw = pltpu.async_copy(
            ew3_hbm.at[w, pl.ds(t * G, G), :], ew_v, sem_i)
        di.wait()
        gathers = []
        for b in range(G):
            gathers.append(pltpu.async_copy(
                y_hbm.at[row_v.at[b]], rows_v.at[b], sem_g.at[b]))
        dc.wait()
        dw.wait()
        for b in range(G):
            gathers[b].wait()

            def scale(g2, _, b=b):
                wvec = ew_v[b, pl.ds(g2 * 16, 16)]
                for i in range(16):
                    wi = wvec[i]
                    r = g2 * 16 + i
                    for j in range(D_HID // 16):
                        sl = pl.ds(j * 16, 16)
                        rows_v[b, r, sl] = rows_v[b, r, sl] * wi
                return 0

            lax.fori_loop(0, CHUNK // 16, scale, 0)
            pltpu.async_copy(
                rows_v.at[b], accum_sh.at[col_v.at[b]], sem_s.at[b],
                add=True)
        return 0

    lax.fori_loop(0, NG, body, 0)
    drain(0, 0)
    plsc.subcore_barrier()

    @pl.when(s == 0)
    def _():
        pltpu.sync_copy(accum_sh, aggp_hbm.at[c])


def _dinv_of(degp_ref):
    deg = degp_ref[0, 0, :] + degp_ref[0, 1, :] + 1.0
    return lax.rsqrt(deg)


def _tc_y1_body(x_ref, w_ref, degp_ref, y_ref):
    dinv = _dinv_of(degp_ref)
    y_ref[...] = jnp.dot(x_ref[...], w_ref[...],
                         preferred_element_type=jnp.float32) * dinv[:, None]


def _tc_y1(x, W1, degp):
    return pl.pallas_call(
        _tc_y1_body,
        grid=(N_BLKS,),
        in_specs=[
            pl.BlockSpec((ROW_BLK, D_IN), lambda i: (i, 0)),
            pl.BlockSpec((D_IN, D_HID), lambda i: (0, 0)),
            pl.BlockSpec((1, NC, ROW_BLK), lambda i: (i, 0, 0)),
        ],
        out_specs=pl.BlockSpec((ROW_BLK, D_HID), lambda i: (i, 0)),
        out_shape=jax.ShapeDtypeStruct((N_NODES, D_HID), jnp.float32),
    )(x, W1, degp)


def _tc_y2_body(aggp_ref, degp_ref, b_ref, w_ref, y_ref):
    dinv = _dinv_of(degp_ref)
    agg = aggp_ref[0, :, :] + aggp_ref[1, :, :]
    h = jnp.maximum(agg * dinv[:, None] + b_ref[...], 0.0)
    y_ref[...] = jnp.dot(h, w_ref[...],
                         preferred_element_type=jnp.float32) * dinv[:, None]


def _tc_y2(aggp, degp, b1, W2):
    return pl.pallas_call(
        _tc_y2_body,
        grid=(N_BLKS,),
        in_specs=[
            pl.BlockSpec((NC, ROW_BLK, D_HID), lambda i: (0, i, 0)),
            pl.BlockSpec((1, NC, ROW_BLK), lambda i: (i, 0, 0)),
            pl.BlockSpec((1, D_HID), lambda i: (0, 0)),
            pl.BlockSpec((D_HID, D_HID), lambda i: (0, 0)),
        ],
        out_specs=pl.BlockSpec((ROW_BLK, D_HID), lambda i: (i, 0)),
        out_shape=jax.ShapeDtypeStruct((N_NODES, D_HID), jnp.float32),
    )(aggp, degp, b1, W2)


def _tc_final_body(aggp_ref, degp_ref, b_ref, batch_ref, wl_ref, bl_ref,
                   out_ref, psum, pcnt):
    i = pl.program_id(0)

    @pl.when(i == 0)
    def _():
        psum[...] = jnp.zeros_like(psum)
        pcnt[...] = jnp.zeros_like(pcnt)

    dinv = _dinv_of(degp_ref)
    agg = aggp_ref[0, :, :] + aggp_ref[1, :, :]
    h = jnp.maximum(agg * dinv[:, None] + b_ref[...], 0.0)
    seg = batch_ref[0, :, :]
    gids = lax.broadcasted_iota(jnp.int32, (N_GRAPHS, ROW_BLK), 0)
    onehot = jnp.where(gids == seg, 1.0, 0.0)
    psum[...] += jnp.dot(onehot, h, preferred_element_type=jnp.float32)
    pcnt[...] += jnp.sum(onehot, axis=1, keepdims=True)

    @pl.when(i == N_BLKS - 1)
    def _():
        pooled = psum[...] / jnp.maximum(pcnt[...], 1.0)
        out_ref[...] = jnp.dot(pooled, wl_ref[...],
                               preferred_element_type=jnp.float32) + bl_ref[...]


def _tc_final(aggp, degp, b2, batch3, Wlin, blin):
    return pl.pallas_call(
        _tc_final_body,
        grid=(N_BLKS,),
        in_specs=[
            pl.BlockSpec((NC, ROW_BLK, D_HID), lambda i: (0, i, 0)),
            pl.BlockSpec((1, NC, ROW_BLK), lambda i: (i, 0, 0)),
            pl.BlockSpec((1, D_HID), lambda i: (0, 0)),
            pl.BlockSpec((1, 1, ROW_BLK), lambda i: (i, 0, 0)),
            pl.BlockSpec((D_HID, D_OUT), lambda i: (0, 0)),
            pl.BlockSpec((1, D_OUT), lambda i: (0, 0)),
        ],
        out_specs=pl.BlockSpec((N_GRAPHS, D_OUT), lambda i: (0, 0)),
        out_shape=jax.ShapeDtypeStruct((N_GRAPHS, D_OUT), jnp.float32),
        scratch_shapes=[
            pltpu.VMEM((N_GRAPHS, D_HID), jnp.float32),
            pltpu.VMEM((N_GRAPHS, 1), jnp.float32),
        ],
    )(aggp, degp, b2, batch3, Wlin, blin)


@jax.jit
def kernel(x, edge_index, edge_attr, batch, W1, b1, W2, b2, Wlin, blin):
    row = edge_index[0].astype(jnp.int32)
    col = edge_index[1].astype(jnp.int32)
    ew = edge_attr.astype(jnp.float32)
    batch3 = batch.astype(jnp.int32).reshape(N_BLKS, 1, ROW_BLK)
    zeros1 = jnp.zeros((N_NODES,), jnp.float32)
    zeros2 = jnp.zeros((N_NODES, D_HID), jnp.float32)

    pad = E_PAD - N_EDGES
    padidx = jnp.arange(pad, dtype=jnp.int32) % N_NODES
    row3 = jnp.concatenate([row, padidx]).reshape(NW, T, CHUNK)
    col3 = jnp.concatenate([col, padidx]).reshape(NW, T, CHUNK)
    ew3 = jnp.concatenate(
        [ew, jnp.zeros((pad,), jnp.float32)]).reshape(NW, T, CHUNK)

    degp = _sc_deg(col3, ew3, zeros1)
    degp3 = degp.reshape(NC, N_BLKS, ROW_BLK).transpose(1, 0, 2)
    y1 = _tc_y1(x, W1, degp3)
    aggp1 = _sc_agg(y1, row3, col3, ew3, zeros2)
    y2 = _tc_y2(aggp1, degp3, b1.reshape(1, D_HID), W2)
    aggp2 = _sc_agg(y2, row3, col3, ew3, zeros2)
    return _tc_final(aggp2, degp3, b2.reshape(1, D_HID), batch3,
                     Wlin, blin.reshape(1, D_OUT))


# speedup vs baseline: 2.4908x; 1.0515x over previous
import functools

import jax
import jax.numpy as jnp
from jax import lax
from jax.experimental import pallas as pl
from jax.experimental.pallas import tpu as pltpu
from jax.experimental.pallas import tpu_sc as plsc

N_NODES = 10000
N_EDGES = 320000
D_IN = 128
D_HID = 128
D_OUT = 10
N_GRAPHS = 64

NC = 2
NS = 16
CHUNK = 32
NW = NC * NS
T = 320
E_PAD = NW * T * CHUNK
G = 8
NG = T // G
CHUNK_D = 128
T_D = E_PAD // (NW * CHUNK_D)
NG_D = T_D // G
ROW_BLK = 1000
N_BLKS = N_NODES // ROW_BLK
ROW_BLK = 1000
N_BLKS = N_NODES // ROW_BLK

_sc_mesh = plsc.VectorSubcoreMesh(
    core_axis_name="c", subcore_axis_name="s", num_cores=NC, num_subcores=NS)


@functools.partial(
    pl.kernel,
    out_type=jax.ShapeDtypeStruct((NC, N_NODES), jnp.float32),
    mesh=_sc_mesh,
    scratch_types=[
        pltpu.VMEM_SHARED((N_NODES,), jnp.float32),
        pltpu.VMEM((G, CHUNK_D), jnp.int32),
        pltpu.VMEM((G, CHUNK_D), jnp.float32),
        pltpu.SemaphoreType.DMA,
        pltpu.SemaphoreType.DMA((G,)),
    ],
)
def _sc_deg(col3_hbm, ew3_hbm, zeros_hbm, degp_hbm,
            deg_sh, col_v, ew_v, sem_i, sem_s):
    c = lax.axis_index("c")
    s = lax.axis_index("s")
    w = c * NS + s

    @pl.when(s == 0)
    def _():
        pltpu.sync_copy(zeros_hbm, deg_sh)

    plsc.subcore_barrier()

    def drain(_t, x):
        for b in range(G):
            pltpu.make_async_copy(
                ew_v.at[b], deg_sh.at[col_v.at[b]], sem_s.at[b]).wait()
        return x

    def body(t, _):
        @pl.when(t > 0)
        def _():
            drain(t, 0)
        di = pltpu.async_copy(
            col3_hbm.at[w, pl.ds(t * G, G), :], col_v, sem_i)
        dw = pltpu.async_copy(
            ew3_hbm.at[w, pl.ds(t * G, G), :], ew_v, sem_i)
        di.wait()
        dw.wait()
        for b in range(G):
            pltpu.async_copy(
                ew_v.at[b], deg_sh.at[col_v.at[b]], sem_s.at[b], add=True)
        return 0

    lax.fori_loop(0, NG_D, body, 0)
    drain(0, 0)
    plsc.subcore_barrier()

    @pl.when(s == 0)
    def _():
        pltpu.sync_copy(deg_sh, degp_hbm.at[c])


@functools.partial(
    pl.kernel,
    out_type=jax.ShapeDtypeStruct((NC, N_NODES, D_HID), jnp.float32),
    mesh=_sc_mesh,
    scratch_types=[
        pltpu.VMEM_SHARED((N_NODES, D_HID), jnp.float32),
        pltpu.VMEM((G, CHUNK), jnp.int32),
        pltpu.VMEM((G, CHUNK), jnp.int32),
        pltpu.VMEM((G, CHUNK), jnp.float32),
        pltpu.VMEM((G, CHUNK, D_HID), jnp.float32),
        pltpu.SemaphoreType.DMA,
        pltpu.SemaphoreType.DMA((G,)),
        pltpu.SemaphoreType.DMA((G,)),
    ],
)
def _sc_agg(y_hbm, row3_hbm, col3_hbm, ew3_hbm, zeros_hbm, aggp_hbm,
            accum_sh, row_v, col_v, ew_v, rows_v, sem_i, sem_g, sem_s):
    c = lax.axis_index("c")
    s = lax.axis_index("s")
    w = c * NS + s

    @pl.when(jnp.logical_and(s == 0, c == 0))
    def _():
        pltpu.sync_copy(y_hbm, accum_sh)

    @pl.when(jnp.logical_and(s == 0, c == 1))
    def _():
        pltpu.sync_copy(zeros_hbm, accum_sh)

    plsc.subcore_barrier()

    def drain(_t, x):
        for b in range(G):
            pltpu.make_async_copy(
                rows_v.at[b], accum_sh.at[col_v.at[b]], sem_s.at[b]).wait()
        return x

    def body(t, _):
        @pl.when(t > 0)
        def _():
            drain(t, 0)
        di = pltpu.async_copy(
            row3_hbm.at[w, pl.ds(t * G, G), :], row_v, sem_i)
        dc = pltpu.async_copy(
            col3_hbm.at[w, pl.ds(t * G, G), :], col_v, sem_i)
        dw = pltpu.async_copy(
            ew3_hbm.at[w, pl.ds(t * G, G), :], ew_v, sem_i)
        di.wait()
        gathers = []
        for b in range(G):
            gathers.append(pltpu.async_copy(
                y_hbm.at[row_v.at[b]], rows_v.at[b], sem_g.at[b]))
        dc.wait()
        dw.wait()
        for b in range(G):
            gathers[b].wait()

            def scale(g2, _, b=b):
                wvec = ew_v[b, pl.ds(g2 * 16, 16)]
                for i in range(16):
                    wi = wvec[i]
                    r = g2 * 16 + i
                    for j in range(D_HID // 16):
                        sl = pl.ds(j * 16, 16)
                        rows_v[b, r, sl] = rows_v[b, r, sl] * wi
                return 0

            lax.fori_loop(0, CHUNK // 16, scale, 0)
            pltpu.async_copy(
                rows_v.at[b], accum_sh.at[col_v.at[b]], sem_s.at[b],
                add=True)
        return 0

    lax.fori_loop(0, NG, body, 0)
    drain(0, 0)
    plsc.subcore_barrier()

    @pl.when(s == 0)
    def _():
        pltpu.sync_copy(accum_sh, aggp_hbm.at[c])


def _dinv_of(degp_ref):
    deg = degp_ref[0, 0, :] + degp_ref[0, 1, :] + 1.0
    return lax.rsqrt(deg)


def _tc_y1_body(x_ref, w_ref, degp_ref, y_ref):
    dinv = _dinv_of(degp_ref)
    y_ref[...] = jnp.dot(x_ref[...], w_ref[...],
                         preferred_element_type=jnp.float32) * dinv[:, None]


def _tc_y1(x, W1, degp):
    return pl.pallas_call(
        _tc_y1_body,
        grid=(N_BLKS,),
        in_specs=[
            pl.BlockSpec((ROW_BLK, D_IN), lambda i: (i, 0)),
            pl.BlockSpec((D_IN, D_HID), lambda i: (0, 0)),
            pl.BlockSpec((1, NC, ROW_BLK), lambda i: (i, 0, 0)),
        ],
        out_specs=pl.BlockSpec((ROW_BLK, D_HID), lambda i: (i, 0)),
        out_shape=jax.ShapeDtypeStruct((N_NODES, D_HID), jnp.float32),
    )(x, W1, degp)


def _tc_y2_body(aggp_ref, degp_ref, b_ref, w_ref, y_ref):
    dinv = _dinv_of(degp_ref)
    agg = aggp_ref[0, :, :] + aggp_ref[1, :, :]
    h = jnp.maximum(agg * dinv[:, None] + b_ref[...], 0.0)
    y_ref[...] = jnp.dot(h, w_ref[...],
                         preferred_element_type=jnp.float32) * dinv[:, None]


def _tc_y2(aggp, degp, b1, W2):
    return pl.pallas_call(
        _tc_y2_body,
        grid=(N_BLKS,),
        in_specs=[
            pl.BlockSpec((NC, ROW_BLK, D_HID), lambda i: (0, i, 0)),
            pl.BlockSpec((1, NC, ROW_BLK), lambda i: (i, 0, 0)),
            pl.BlockSpec((1, D_HID), lambda i: (0, 0)),
            pl.BlockSpec((D_HID, D_HID), lambda i: (0, 0)),
        ],
        out_specs=pl.BlockSpec((ROW_BLK, D_HID), lambda i: (i, 0)),
        out_shape=jax.ShapeDtypeStruct((N_NODES, D_HID), jnp.float32),
    )(aggp, degp, b1, W2)


def _tc_final_body(aggp_ref, degp_ref, b_ref, batch_ref, wl_ref, bl_ref,
                   out_ref, psum, pcnt):
    i = pl.program_id(0)

    @pl.when(i == 0)
    def _():
        psum[...] = jnp.zeros_like(psum)
        pcnt[...] = jnp.zeros_like(pcnt)

    dinv = _dinv_of(degp_ref)
    agg = aggp_ref[0, :, :] + aggp_ref[1, :, :]
    h = jnp.maximum(agg * dinv[:, None] + b_ref[...], 0.0)
    seg = batch_ref[0, :, :]
    gids = lax.broadcasted_iota(jnp.int32, (N_GRAPHS, ROW_BLK), 0)
    onehot = jnp.where(gids == seg, 1.0, 0.0)
    psum[...] += jnp.dot(onehot, h, preferred_element_type=jnp.float32)
    pcnt[...] += jnp.sum(onehot, axis=1, keepdims=True)

    @pl.when(i == N_BLKS - 1)
    def _():
        pooled = psum[...] / jnp.maximum(pcnt[...], 1.0)
        out_ref[...] = jnp.dot(pooled, wl_ref[...],
                               preferred_element_type=jnp.float32) + bl_ref[...]


def _tc_final(aggp, degp, b2, batch3, Wlin, blin):
    return pl.pallas_call(
        _tc_final_body,
        grid=(N_BLKS,),
        in_specs=[
            pl.BlockSpec((NC, ROW_BLK, D_HID), lambda i: (0, i, 0)),
            pl.BlockSpec((1, NC, ROW_BLK), lambda i: (i, 0, 0)),
            pl.BlockSpec((1, D_HID), lambda i: (0, 0)),
            pl.BlockSpec((1, 1, ROW_BLK), lambda i: (i, 0, 0)),
            pl.BlockSpec((D_HID, D_OUT), lambda i: (0, 0)),
            pl.BlockSpec((1, D_OUT), lambda i: (0, 0)),
        ],
        out_specs=pl.BlockSpec((N_GRAPHS, D_OUT), lambda i: (0, 0)),
        out_shape=jax.ShapeDtypeStruct((N_GRAPHS, D_OUT), jnp.float32),
        scratch_shapes=[
            pltpu.VMEM((N_GRAPHS, D_HID), jnp.float32),
            pltpu.VMEM((N_GRAPHS, 1), jnp.float32),
        ],
    )(aggp, degp, b2, batch3, Wlin, blin)


@jax.jit
def kernel(x, edge_index, edge_attr, batch, W1, b1, W2, b2, Wlin, blin):
    row = edge_index[0].astype(jnp.int32)
    col = edge_index[1].astype(jnp.int32)
    ew = edge_attr.astype(jnp.float32)
    batch3 = batch.astype(jnp.int32).reshape(N_BLKS, 1, ROW_BLK)
    zeros1 = jnp.zeros((N_NODES,), jnp.float32)
    zeros2 = jnp.zeros((N_NODES, D_HID), jnp.float32)

    pad = E_PAD - N_EDGES
    padidx = jnp.arange(pad, dtype=jnp.int32) % N_NODES
    rowp = jnp.concatenate([row, padidx])
    colp = jnp.concatenate([col, padidx])
    ewp = jnp.concatenate([ew, jnp.zeros((pad,), jnp.float32)])
    row3 = rowp.reshape(NW, T, CHUNK)
    col3 = colp.reshape(NW, T, CHUNK)
    ew3 = ewp.reshape(NW, T, CHUNK)
    col3d = colp.reshape(NW, T_D, CHUNK_D)
    ew3d = ewp.reshape(NW, T_D, CHUNK_D)

    degp = _sc_deg(col3d, ew3d, zeros1)
    degp3 = degp.reshape(NC, N_BLKS, ROW_BLK).transpose(1, 0, 2)
    y1 = _tc_y1(x, W1, degp3)
    aggp1 = _sc_agg(y1, row3, col3, ew3, zeros2)
    y2 = _tc_y2(aggp1, degp3, b1.reshape(1, D_HID), W2)
    aggp2 = _sc_agg(y2, row3, col3, ew3, zeros2)
    return _tc_final(aggp2, degp3, b2.reshape(1, D_HID), batch3,
                     Wlin, blin.reshape(1, D_OUT))


# speedup vs baseline: 2.4934x; 1.0011x over previous
import functools

import jax
import jax.numpy as jnp
from jax import lax
from jax.experimental import pallas as pl
from jax.experimental.pallas import tpu as pltpu
from jax.experimental.pallas import tpu_sc as plsc

N_NODES = 10000
N_EDGES = 320000
D_IN = 128
D_HID = 128
D_OUT = 10
N_GRAPHS = 64

NC = 2
NS = 16
CHUNK = 32
NW = NC * NS
T = 320
E_PAD = NW * T * CHUNK
G = 8
NG = T // G
CHUNK_D = 128
E_PAD_D = 327680
T_D = E_PAD_D // (NW * CHUNK_D)
NG_D = T_D // G
ROW_BLK = 1000
N_BLKS = N_NODES // ROW_BLK
ROW_BLK = 1000
N_BLKS = N_NODES // ROW_BLK

_sc_mesh = plsc.VectorSubcoreMesh(
    core_axis_name="c", subcore_axis_name="s", num_cores=NC, num_subcores=NS)


@functools.partial(
    pl.kernel,
    out_type=jax.ShapeDtypeStruct((NC, N_NODES), jnp.float32),
    mesh=_sc_mesh,
    scratch_types=[
        pltpu.VMEM_SHARED((N_NODES,), jnp.float32),
        pltpu.VMEM((G, CHUNK_D), jnp.int32),
        pltpu.VMEM((G, CHUNK_D), jnp.float32),
        pltpu.SemaphoreType.DMA,
        pltpu.SemaphoreType.DMA((G,)),
    ],
)
def _sc_deg(col3_hbm, ew3_hbm, zeros_hbm, degp_hbm,
            deg_sh, col_v, ew_v, sem_i, sem_s):
    c = lax.axis_index("c")
    s = lax.axis_index("s")
    w = c * NS + s

    @pl.when(s == 0)
    def _():
        pltpu.sync_copy(zeros_hbm, deg_sh)

    plsc.subcore_barrier()

    def drain(_t, x):
        for b in range(G):
            pltpu.make_async_copy(
                ew_v.at[b], deg_sh.at[col_v.at[b]], sem_s.at[b]).wait()
        return x

    def body(t, _):
        @pl.when(t > 0)
        def _():
            drain(t, 0)
        di = pltpu.async_copy(
            col3_hbm.at[w, pl.ds(t * G, G), :], col_v, sem_i)
        dw = pltpu.async_copy(
            ew3_hbm.at[w, pl.ds(t * G, G), :], ew_v, sem_i)
        di.wait()
        dw.wait()
        for b in range(G):
            pltpu.async_copy(
                ew_v.at[b], deg_sh.at[col_v.at[b]], sem_s.at[b], add=True)
        return 0

    lax.fori_loop(0, NG_D, body, 0)
    drain(0, 0)
    plsc.subcore_barrier()

    @pl.when(s == 0)
    def _():
        pltpu.sync_copy(deg_sh, degp_hbm.at[c])


@functools.partial(
    pl.kernel,
    out_type=jax.ShapeDtypeStruct((NC, N_NODES, D_HID), jnp.float32),
    mesh=_sc_mesh,
    scratch_types=[
        pltpu.VMEM_SHARED((N_NODES, D_HID), jnp.float32),
        pltpu.VMEM((G, CHUNK), jnp.int32),
        pltpu.VMEM((G, CHUNK), jnp.int32),
        pltpu.VMEM((G, CHUNK), jnp.float32),
        pltpu.VMEM((G, CHUNK, D_HID), jnp.float32),
        pltpu.SemaphoreType.DMA,
        pltpu.SemaphoreType.DMA((G,)),
        pltpu.SemaphoreType.DMA((G,)),
    ],
)
def _sc_agg(y_hbm, row3_hbm, col3_hbm, ew3_hbm, zeros_hbm, aggp_hbm,
            accum_sh, row_v, col_v, ew_v, rows_v, sem_i, sem_g, sem_s):
    c = lax.axis_index("c")
    s = lax.axis_index("s")
    w = c * NS + s

    @pl.when(jnp.logical_and(s == 0, c == 0))
    def _():
        pltpu.sync_copy(y_hbm, accum_sh)

    @pl.when(jnp.logical_and(s == 0, c == 1))
    def _():
        pltpu.sync_copy(zeros_hbm, accum_sh)

    plsc.subcore_barrier()

    def drain(_t, x):
        for b in range(G):
            pltpu.make_async_copy(
                rows_v.at[b], accum_sh.at[col_v.at[b]], sem_s.at[b]).wait()
        return x

    def body(t, _):
        @pl.when(t > 0)
        def _():
            drain(t, 0)
        di = pltpu.async_copy(
            row3_hbm.at[w, pl.ds(t * G, G), :], row_v, sem_i)
        dc = pltpu.async_copy(
            col3_hbm.at[w, pl.ds(t * G, G), :], col_v, sem_i)
        dw = pltpu.async_copy(
            ew3_hbm.at[w, pl.ds(t * G, G), :], ew_v, sem_i)
        di.wait()
        gathers = []
        for b in range(G):
            gathers.append(pltpu.async_copy(
                y_hbm.at[row_v.at[b]], rows_v.at[b], sem_g.at[b]))
        dc.wait()
        dw.wait()
        for b in range(G):
            gathers[b].wait()

            def scale(g2, _, b=b):
                wvec = ew_v[b, pl.ds(g2 * 16, 16)]
                for i in range(16):
                    wi = wvec[i]
                    r = g2 * 16 + i
                    for j in range(D_HID // 16):
                        sl = pl.ds(j * 16, 16)
                        rows_v[b, r, sl] = rows_v[b, r, sl] * wi
                return 0

            lax.fori_loop(0, CHUNK // 16, scale, 0)
            pltpu.async_copy(
                rows_v.at[b], accum_sh.at[col_v.at[b]], sem_s.at[b],
                add=True)
        return 0

    lax.fori_loop(0, NG, body, 0)
    drain(0, 0)
    plsc.subcore_barrier()

    @pl.when(s == 0)
    def _():
        pltpu.sync_copy(accum_sh, aggp_hbm.at[c])


def _dinv_of(degp_ref):
    deg = degp_ref[0, 0, :] + degp_ref[0, 1, :] + 1.0
    return lax.rsqrt(deg)


def _tc_y1_body(x_ref, w_ref, degp_ref, y_ref):
    dinv = _dinv_of(degp_ref)
    y_ref[...] = jnp.dot(x_ref[...], w_ref[...],
                         preferred_element_type=jnp.float32) * dinv[:, None]


def _tc_y1(x, W1, degp):
    return pl.pallas_call(
        _tc_y1_body,
        grid=(N_BLKS,),
        in_specs=[
            pl.BlockSpec((ROW_BLK, D_IN), lambda i: (i, 0)),
            pl.BlockSpec((D_IN, D_HID), lambda i: (0, 0)),
            pl.BlockSpec((1, NC, ROW_BLK), lambda i: (i, 0, 0)),
        ],
        out_specs=pl.BlockSpec((ROW_BLK, D_HID), lambda i: (i, 0)),
        out_shape=jax.ShapeDtypeStruct((N_NODES, D_HID), jnp.float32),
    )(x, W1, degp)


def _tc_y2_body(aggp_ref, degp_ref, b_ref, w_ref, y_ref):
    dinv = _dinv_of(degp_ref)
    agg = aggp_ref[0, :, :] + aggp_ref[1, :, :]
    h = jnp.maximum(agg * dinv[:, None] + b_ref[...], 0.0)
    y_ref[...] = jnp.dot(h, w_ref[...],
                         preferred_element_type=jnp.float32) * dinv[:, None]


def _tc_y2(aggp, degp, b1, W2):
    return pl.pallas_call(
        _tc_y2_body,
        grid=(N_BLKS,),
        in_specs=[
            pl.BlockSpec((NC, ROW_BLK, D_HID), lambda i: (0, i, 0)),
            pl.BlockSpec((1, NC, ROW_BLK), lambda i: (i, 0, 0)),
            pl.BlockSpec((1, D_HID), lambda i: (0, 0)),
            pl.BlockSpec((D_HID, D_HID), lambda i: (0, 0)),
        ],
        out_specs=pl.BlockSpec((ROW_BLK, D_HID), lambda i: (i, 0)),
        out_shape=jax.ShapeDtypeStruct((N_NODES, D_HID), jnp.float32),
    )(aggp, degp, b1, W2)


def _tc_final_body(aggp_ref, degp_ref, b_ref, batch_ref, wl_ref, bl_ref,
                   out_ref, psum, pcnt):
    i = pl.program_id(0)

    @pl.when(i == 0)
    def _():
        psum[...] = jnp.zeros_like(psum)
        pcnt[...] = jnp.zeros_like(pcnt)

    dinv = _dinv_of(degp_ref)
    agg = aggp_ref[0, :, :] + aggp_ref[1, :, :]
    h = jnp.maximum(agg * dinv[:, None] + b_ref[...], 0.0)
    seg = batch_ref[0, :, :]
    gids = lax.broadcasted_iota(jnp.int32, (N_GRAPHS, ROW_BLK), 0)
    onehot = jnp.where(gids == seg, 1.0, 0.0)
    psum[...] += jnp.dot(onehot, h, preferred_element_type=jnp.float32)
    pcnt[...] += jnp.sum(onehot, axis=1, keepdims=True)

    @pl.when(i == N_BLKS - 1)
    def _():
        pooled = psum[...] / jnp.maximum(pcnt[...], 1.0)
        out_ref[...] = jnp.dot(pooled, wl_ref[...],
                               preferred_element_type=jnp.float32) + bl_ref[...]


def _tc_final(aggp, degp, b2, batch3, Wlin, blin):
    return pl.pallas_call(
        _tc_final_body,
        grid=(N_BLKS,),
        in_specs=[
            pl.BlockSpec((NC, ROW_BLK, D_HID), lambda i: (0, i, 0)),
            pl.BlockSpec((1, NC, ROW_BLK), lambda i: (i, 0, 0)),
            pl.BlockSpec((1, D_HID), lambda i: (0, 0)),
            pl.BlockSpec((1, 1, ROW_BLK), lambda i: (i, 0, 0)),
            pl.BlockSpec((D_HID, D_OUT), lambda i: (0, 0)),
            pl.BlockSpec((1, D_OUT), lambda i: (0, 0)),
        ],
        out_specs=pl.BlockSpec((N_GRAPHS, D_OUT), lambda i: (0, 0)),
        out_shape=jax.ShapeDtypeStruct((N_GRAPHS, D_OUT), jnp.float32),
        scratch_shapes=[
            pltpu.VMEM((N_GRAPHS, D_HID), jnp.float32),
            pltpu.VMEM((N_GRAPHS, 1), jnp.float32),
        ],
    )(aggp, degp, b2, batch3, Wlin, blin)


@jax.jit
def kernel(x, edge_index, edge_attr, batch, W1, b1, W2, b2, Wlin, blin):
    row = edge_index[0].astype(jnp.int32)
    col = edge_index[1].astype(jnp.int32)
    ew = edge_attr.astype(jnp.float32)
    batch3 = batch.astype(jnp.int32).reshape(N_BLKS, 1, ROW_BLK)
    zeros1 = jnp.zeros((N_NODES,), jnp.float32)
    zeros2 = jnp.zeros((N_NODES, D_HID), jnp.float32)

    pad = E_PAD - N_EDGES
    padidx = jnp.arange(pad, dtype=jnp.int32) % N_NODES
    row3 = jnp.concatenate([row, padidx]).reshape(NW, T, CHUNK)
    col3 = jnp.concatenate([col, padidx]).reshape(NW, T, CHUNK)
    ew3 = jnp.concatenate(
        [ew, jnp.zeros((pad,), jnp.float32)]).reshape(NW, T, CHUNK)
    pad_d = E_PAD_D - N_EDGES
    padidx_d = jnp.arange(pad_d, dtype=jnp.int32) % N_NODES
    col3d = jnp.concatenate([col, padidx_d]).reshape(NW, T_D, CHUNK_D)
    ew3d = jnp.concatenate(
        [ew, jnp.zeros((pad_d,), jnp.float32)]).reshape(NW, T_D, CHUNK_D)

    degp = _sc_deg(col3d, ew3d, zeros1)
    degp3 = degp.reshape(NC, N_BLKS, ROW_BLK).transpose(1, 0, 2)
    y1 = _tc_y1(x, W1, degp3)
    aggp1 = _sc_agg(y1, row3, col3, ew3, zeros2)
    y2 = _tc_y2(aggp1, degp3, b1.reshape(1, D_HID), W2)
    aggp2 = _sc_agg(y2, row3, col3, ew3, zeros2)
    return _tc_final(aggp2, degp3, b2.reshape(1, D_HID), batch3,
                     Wlin, blin.reshape(1, D_OUT))
